# baseline (speedup 1.0000x reference)
import dataclasses
import functools

import jax
import jax.numpy as jnp
from jax import lax
from jax.experimental import pallas as pl
from jax.experimental.pallas import tpu as pltpu
from jax.experimental.pallas import tpu_sc as plsc

_NC = 2
_NS = 16
_L = 16
_NW = _NC * _NS

_W = 32
_NB = 8


def _pack_table(feats32):
    RT, C = feats32.shape
    C2 = C // 2
    RB = 4096

    def body(x_ref, o_ref):
        ua = jax.lax.bitcast_convert_type(x_ref[:, :C2], jnp.uint32)
        ub = jax.lax.bitcast_convert_type(x_ref[:, C2:], jnp.uint32)
        ra = (ua + 0x7FFF + ((ua >> 16) & 1)) >> 16
        rb = (ub + 0x7FFF + ((ub >> 16) & 1)) >> 16
        o_ref[...] = jax.lax.bitcast_convert_type(ra | (rb << 16), jnp.int32)

    return pl.pallas_call(
        body,
        out_shape=jax.ShapeDtypeStruct((RT, C2), jnp.int32),
        grid=(RT // RB,),
        in_specs=[pl.BlockSpec((RB, C), lambda i: (i, 0))],
        out_specs=pl.BlockSpec((RB, C2), lambda i: (i, 0)),
    )(feats32)


def _sc_interpolate(feats, planes, M, N, K, C):
    P = M // _NW
    T = P // _W
    C2 = C // 2
    mesh = plsc.VectorSubcoreMesh(core_axis_name="c", subcore_axis_name="s")
    cp = pltpu.CompilerParams()
    if "needs_layout_passes" in pltpu.CompilerParams.__dataclass_fields__:
        cp = dataclasses.replace(cp, needs_layout_passes=False)

    @functools.partial(
        pl.kernel,
        out_type=jax.ShapeDtypeStruct((M, C), jnp.float32),
        mesh=mesh,
        scratch_types=[
            pltpu.VMEM((K, P), jnp.int32),
            pltpu.VMEM((K, P), jnp.int32),
            *[pltpu.VMEM((K * _W, C2), jnp.int32)
              for _ in range(_NB)],
            pltpu.VMEM((_W, C), jnp.float32),
            pltpu.VMEM((_W, C), jnp.float32),
            *[pltpu.SemaphoreType.DMA for _ in range(_NB)],
            pltpu.SemaphoreType.DMA,
            pltpu.SemaphoreType.DMA,
        ],
        compiler_params=cp,
    )
    def body(feats_hbm, p0_hbm, p1_hbm, p2_hbm, out_hbm,
             pw, idxs, rows0, rows1, rows2, rows3, rows4, rows5, rows6, rows7,
             outv0, outv1, sg0, sg1, sg2, sg3, sg4, sg5, sg6, sg7, so0, so1):
        wid = lax.axis_index("s") * _NC + lax.axis_index("c")
        base_pt = wid * P
        bb = base_pt // N
        n0 = base_pt - bb * N
        rows = (rows0, rows1, rows2, rows3, rows4, rows5, rows6, rows7)
        outv = (outv0, outv1)
        sg = (sg0, sg1, sg2, sg3, sg4, sg5, sg6, sg7)
        so = (so0, so1)

        for k, p_hbm in enumerate((p0_hbm, p1_hbm, p2_hbm)):
            pltpu.sync_copy(p_hbm.at[pl.ds(bb, 1), pl.ds(n0, P)],
                            pw.at[pl.ds(k, 1)])

        @plsc.parallel_loop(0, P, step=_L)
        def _gl(j):
            for k in range(K):
                idxs[k, pl.ds(j, _L)] = pw[k, pl.ds(j, _L)] & 0xFFFF

        def start_gather(t, b):
            for k in range(K):
                pltpu.async_copy(
                    feats_hbm.at[idxs.at[k, pl.ds(t * _W, _W)]],
                    rows[b].at[pl.ds(k * _W, _W)], sg[b])

        def wait_gather(t, b):
            for k in range(K):
                pltpu.make_async_copy(
                    feats_hbm.at[idxs.at[k, pl.ds(t * _W, _W)]],
                    rows[b].at[pl.ds(k * _W, _W)], sg[b]).wait()

        def compute(t, rb, b):
            @plsc.parallel_loop(0, _W, unroll=2)
            def _pt(i):
                pt = jnp.full((_L,), t * _W + i, dtype=jnp.int32)
                wbs = []
                for k in range(K):
                    spw = plsc.load_gather(
                        pw, [jnp.full((_L,), k, dtype=jnp.int32), pt])
                    hi = spw & jnp.int32(-65536)
                    both = hi | lax.shift_right_logical(hi, 16)
                    wbs.append(plsc.bitcast(both, jnp.bfloat16))
                for c in range(C2 // _L):
                    sl = pl.ds(c * _L, _L)
                    acc = plsc.bitcast(rows[rb][i, sl], jnp.bfloat16) * wbs[0]
                    for k in range(1, K):
                        acc += plsc.bitcast(rows[rb][k * _W + i, sl],
                                            jnp.bfloat16) * wbs[k]
                    lo, hi2 = plsc.unpack(acc, format=plsc.PackFormat.INTERLEAVED)
                    outv[b][i, sl] = lo
                    outv[b][i, pl.ds(C2 + c * _L, _L)] = hi2

        for b in range(_NB):
            start_gather(b, b)

        @pl.loop(0, T, step=_NB)
        def _win(t):
            for b in range(_NB):
                tt = t + b
                ob = b % 2
                wait_gather(tt, b)
                @pl.when(tt >= 2)
                def _():
                    pltpu.make_async_copy(
                        outv[ob],
                        out_hbm.at[pl.ds(base_pt + (tt - 2) * _W, _W)],
                        so[ob]).wait()
                compute(tt, b, ob)
                pltpu.async_copy(
                    outv[ob],
                    out_hbm.at[pl.ds(base_pt + tt * _W, _W)], so[ob])
                @pl.when(tt + _NB < T)
                def _():
                    start_gather(tt + _NB, b)

        for b in range(2):
            pltpu.make_async_copy(
                outv[b],
                out_hbm.at[pl.ds(base_pt + (T - 2 + b) * _W, _W)],
                so[b]).wait()

    return body(feats, *planes)


def kernel(features, idx, weight):
    B, N, K = idx.shape
    R, C = features.shape[1], features.shape[2]
    M = B * N
    feats = _pack_table(features.reshape(B * R, C))
    wu = jax.lax.bitcast_convert_type(weight, jnp.uint32)
    wbits = (wu + 0x7FFF + ((wu >> 16) & 1)) & jnp.uint32(0xFFFF0000)
    gidx = (idx.astype(jnp.uint32)
            + (jnp.arange(B, dtype=jnp.uint32) * R)[:, None, None])
    word = jax.lax.bitcast_convert_type(wbits | gidx, jnp.int32)
    planes = [word[:, :, k] for k in range(K)]
    out = _sc_interpolate(feats, planes, M, N, K, C)
    return out.reshape(B, N, C)


# speedup vs baseline: 1.0361x; 1.0361x over previous
import dataclasses
import functools

import jax
import jax.numpy as jnp
from jax import lax
from jax.experimental import pallas as pl
from jax.experimental.pallas import tpu as pltpu
from jax.experimental.pallas import tpu_sc as plsc

_NC = 2
_NS = 16
_L = 16
_NW = _NC * _NS

_W = 32
_NB = 4


def _pack_table(feats32):
    RT, C = feats32.shape
    C2 = C // 2
    RB = 4096

    def body(x_ref, o_ref):
        ua = jax.lax.bitcast_convert_type(x_ref[:, :C2], jnp.uint32)
        ub = jax.lax.bitcast_convert_type(x_ref[:, C2:], jnp.uint32)
        ra = (ua + 0x7FFF + ((ua >> 16) & 1)) >> 16
        rb = (ub + 0x7FFF + ((ub >> 16) & 1)) >> 16
        o_ref[...] = jax.lax.bitcast_convert_type(ra | (rb << 16), jnp.int32)

    return pl.pallas_call(
        body,
        out_shape=jax.ShapeDtypeStruct((RT, C2), jnp.int32),
        grid=(RT // RB,),
        in_specs=[pl.BlockSpec((RB, C), lambda i: (i, 0))],
        out_specs=pl.BlockSpec((RB, C2), lambda i: (i, 0)),
    )(feats32)


def _sc_interpolate(feats, planes, M, N, K, C):
    P = M // _NW
    T = P // _W
    C2 = C // 2
    mesh = plsc.VectorSubcoreMesh(core_axis_name="c", subcore_axis_name="s")
    cp = pltpu.CompilerParams()
    if "needs_layout_passes" in pltpu.CompilerParams.__dataclass_fields__:
        cp = dataclasses.replace(cp, needs_layout_passes=False)

    @functools.partial(
        pl.kernel,
        out_type=jax.ShapeDtypeStruct((M, C), jnp.float32),
        mesh=mesh,
        scratch_types=[
            pltpu.VMEM((K, P), jnp.int32),
            pltpu.VMEM((K, P), jnp.int32),
            *[pltpu.VMEM((K * _W, C2), jnp.int32)
              for _ in range(_NB)],
            pltpu.VMEM((_W, C), jnp.float32),
            pltpu.VMEM((_W, C), jnp.float32),
            *[pltpu.SemaphoreType.DMA for _ in range(_NB)],
            pltpu.SemaphoreType.DMA,
            pltpu.SemaphoreType.DMA,
        ],
        compiler_params=cp,
    )
    def body(feats_hbm, p0_hbm, p1_hbm, p2_hbm, out_hbm,
             pw, idxs, rows0, rows1, rows2, rows3, outv0, outv1,
             sg0, sg1, sg2, sg3, so0, so1):
        wid = lax.axis_index("s") * _NC + lax.axis_index("c")
        base_pt = wid * P
        bb = base_pt // N
        n0 = base_pt - bb * N
        rows = (rows0, rows1, rows2, rows3)
        outv = (outv0, outv1)
        sg = (sg0, sg1, sg2, sg3)
        so = (so0, so1)

        for k, p_hbm in enumerate((p0_hbm, p1_hbm, p2_hbm)):
            pltpu.sync_copy(p_hbm.at[pl.ds(bb, 1), pl.ds(n0, P)],
                            pw.at[pl.ds(k, 1)])

        @plsc.parallel_loop(0, P, step=_L)
        def _gl(j):
            for k in range(K):
                idxs[k, pl.ds(j, _L)] = pw[k, pl.ds(j, _L)] & 0xFFFF

        def start_gather(t, b):
            for k in range(K):
                pltpu.async_copy(
                    feats_hbm.at[idxs.at[k, pl.ds(t * _W, _W)]],
                    rows[b].at[pl.ds(k * _W, _W)], sg[b])

        def wait_gather(t, b):
            for k in range(K):
                pltpu.make_async_copy(
                    feats_hbm.at[idxs.at[k, pl.ds(t * _W, _W)]],
                    rows[b].at[pl.ds(k * _W, _W)], sg[b]).wait()

        def compute(t, rb, b):
            @plsc.parallel_loop(0, _W, unroll=2)
            def _pt(i):
                pt = jnp.full((_L,), t * _W + i, dtype=jnp.int32)
                wbs = []
                for k in range(K):
                    spw = plsc.load_gather(
                        pw, [jnp.full((_L,), k, dtype=jnp.int32), pt])
                    hi = spw & jnp.int32(-65536)
                    both = hi | lax.shift_right_logical(hi, 16)
                    wbs.append(plsc.bitcast(both, jnp.bfloat16))
                for c in range(C2 // _L):
                    sl = pl.ds(c * _L, _L)
                    acc = plsc.bitcast(rows[rb][i, sl], jnp.bfloat16) * wbs[0]
                    for k in range(1, K):
                        acc += plsc.bitcast(rows[rb][k * _W + i, sl],
                                            jnp.bfloat16) * wbs[k]
                    lo, hi2 = plsc.unpack(acc, format=plsc.PackFormat.INTERLEAVED)
                    outv[b][i, sl] = lo
                    outv[b][i, pl.ds(C2 + c * _L, _L)] = hi2

        for b in range(_NB):
            start_gather(b, b)

        @pl.loop(0, T, step=_NB)
        def _win(t):
            for b in range(_NB):
                tt = t + b
                ob = b % 2
                wait_gather(tt, b)
                @pl.when(tt >= 2)
                def _():
                    pltpu.make_async_copy(
                        outv[ob],
                        out_hbm.at[pl.ds(base_pt + (tt - 2) * _W, _W)],
                        so[ob]).wait()
                compute(tt, b, ob)
                pltpu.async_copy(
                    outv[ob],
                    out_hbm.at[pl.ds(base_pt + tt * _W, _W)], so[ob])
                @pl.when(tt + _NB < T)
                def _():
                    start_gather(tt + _NB, b)

        for b in range(2):
            pltpu.make_async_copy(
                outv[b],
                out_hbm.at[pl.ds(base_pt + (T - 2 + b) * _W, _W)],
                so[b]).wait()

    return body(feats, *planes)


def kernel(features, idx, weight):
    B, N, K = idx.shape
    R, C = features.shape[1], features.shape[2]
    M = B * N
    feats = _pack_table(features.reshape(B * R, C))
    wu = jax.lax.bitcast_convert_type(weight, jnp.uint32)
    wbits = (wu + 0x7FFF + ((wu >> 16) & 1)) & jnp.uint32(0xFFFF0000)
    gidx = (idx.astype(jnp.uint32)
            + (jnp.arange(B, dtype=jnp.uint32) * R)[:, None, None])
    word = jax.lax.bitcast_convert_type(wbits | gidx, jnp.int32)
    planes = [word[:, :, k] for k in range(K)]
    out = _sc_interpolate(feats, planes, M, N, K, C)
    return out.reshape(B, N, C)
